# Initial kernel scaffold; baseline (speedup 1.0000x reference)
import jax
import jax.numpy as jnp
from jax.experimental import pallas as pl


def kernel(scalar_features, vector_features, edge_index, edge_vec, edge_dist, centers, widths, W_r0, b_r0, W_r1, b_r1, W_r2, b_r2, W_s0, b_s0, W_s1, b_s1, W_vl, W_u0, b_u0, W_u1, b_u1, W_vu, g_sn, b_sn, g_vn, b_vn):
    raise NotImplementedError("write your pallas kernel here")


# speedup vs baseline: 10.3612x; 10.3612x over previous
import functools

import jax
import jax.numpy as jnp
import numpy as np
from jax import lax
from jax.experimental import pallas as pl
from jax.experimental.pallas import tpu as pltpu
from jax.experimental.pallas import tpu_sc as plsc

_CUTOFF = 10.0
_NW = 32
_CH = 128


def _silu(x):
    return x / (1.0 + jnp.exp(-x))


def _ln(x, g, b, eps=1e-5):
    mu = jnp.mean(x, axis=-1, keepdims=True)
    xc = x - mu
    var = jnp.mean(xc * xc, axis=-1, keepdims=True)
    return xc * lax.rsqrt(var + eps) * g + b


def _vl_body(v3_ref, wvl_ref, out_ref):
    w = wvl_ref[...]
    out_ref[...] = jnp.concatenate(
        [jnp.dot(v3_ref[k], w, preferred_element_type=jnp.float32)
         for k in range(3)], axis=1)


def _vl_transform(v3, w_vl, bn):
    n = v3.shape[1]
    h = v3.shape[2]
    return pl.pallas_call(
        _vl_body,
        grid=(n // bn,),
        in_specs=[
            pl.BlockSpec((3, bn, h), lambda i: (0, i, 0)),
            pl.BlockSpec((h, h), lambda i: (0, 0)),
        ],
        out_specs=pl.BlockSpec((bn, 3 * h), lambda i: (i, 0)),
        out_shape=jax.ShapeDtypeStruct((n, 3 * h), jnp.float32),
    )(v3, w_vl)


def _gather_body(sp, vpf, src2, dst2, o_ssrc, o_sdst, o_vps,
                 isrc, idst, r1, r2, r3, sem1, sem2, sem3):
    rows = src2.shape[0] // _NW
    wid = lax.axis_index("s") * 2 + lax.axis_index("c")
    row0 = wid * rows
    pltpu.sync_copy(src2.at[pl.ds(row0, rows)], isrc)
    pltpu.sync_copy(dst2.at[pl.ds(row0, rows)], idst)

    def chunk(i, carry):
        e0 = (row0 + i) * _CH
        c1 = pltpu.async_copy(sp.at[isrc.at[i]], r1, sem1)
        c2 = pltpu.async_copy(sp.at[idst.at[i]], r2, sem2)
        c3 = pltpu.async_copy(vpf.at[isrc.at[i]], r3, sem3)
        c1.wait()
        c2.wait()
        c3.wait()
        pltpu.sync_copy(r1, o_ssrc.at[pl.ds(e0, _CH)])
        pltpu.sync_copy(r2, o_sdst.at[pl.ds(e0, _CH)])
        pltpu.sync_copy(r3, o_vps.at[pl.ds(e0, _CH)])
        return carry

    lax.fori_loop(0, rows, chunk, 0)


def _sc_gather(sp, vpf, src2, dst2):
    n, h = sp.shape
    ep = src2.shape[0] * _CH
    rows = src2.shape[0] // _NW
    mesh = plsc.VectorSubcoreMesh(core_axis_name="c", subcore_axis_name="s", num_cores=2, num_subcores=16)
    f = functools.partial(
        pl.kernel,
        out_type=[
            jax.ShapeDtypeStruct((ep, h), jnp.float32),
            jax.ShapeDtypeStruct((ep, h), jnp.float32),
            jax.ShapeDtypeStruct((ep, 3 * h), jnp.float32),
        ],
        mesh=mesh,
        scratch_types=[
            pltpu.VMEM((rows, _CH), jnp.int32),
            pltpu.VMEM((rows, _CH), jnp.int32),
            pltpu.VMEM((_CH, h), jnp.float32),
            pltpu.VMEM((_CH, h), jnp.float32),
            pltpu.VMEM((_CH, 3 * h), jnp.float32),
            pltpu.SemaphoreType.DMA,
            pltpu.SemaphoreType.DMA,
            pltpu.SemaphoreType.DMA,
        ],
    )(_gather_body)
    return f(sp, vpf, src2, dst2)


def _edge_body(d_ref, ev_ref, ssrc_ref, sdst_ref, vps_ref,
               cen_ref, wid_ref, wr0_ref, br0_ref, wr1_ref, br1_ref,
               wr2_ref, br2_ref, ws0a_ref, ws0b_ref, bs0_ref,
               ws1_ref, bs1_ref, out_ref):
    d = d_ref[...]
    t = (d - cen_ref[...]) / wid_ref[...]
    rbf = jnp.exp(-(t * t))
    cw = 0.5 * (jnp.cos((np.pi / _CUTOFF) * d) + 1.0)
    cw = cw * (d < _CUTOFF).astype(jnp.float32)

    h = _silu(jnp.dot(rbf, wr0_ref[...], preferred_element_type=jnp.float32)
              + br0_ref[...])
    h = _silu(jnp.dot(h, wr1_ref[...], preferred_element_type=jnp.float32)
              + br1_ref[...])
    radial = (jnp.dot(h, wr2_ref[...], preferred_element_type=jnp.float32)
              + br2_ref[...]) * cw

    ssrc = ssrc_ref[...]
    m = _silu(jnp.dot(ssrc, ws0a_ref[...], preferred_element_type=jnp.float32)
              + jnp.dot(sdst_ref[...], ws0b_ref[...],
                        preferred_element_type=jnp.float32)
              + bs0_ref[...])
    smsg = (jnp.dot(m, ws1_ref[...], preferred_element_type=jnp.float32)
            + bs1_ref[...]) * radial

    ev = ev_ref[...]
    vps = vps_ref[...]
    hh = ssrc.shape[1]
    vm = [radial * (vps[:, k * hh:(k + 1) * hh] + ev[:, k:k + 1] * ssrc)
          for k in range(3)]
    out_ref[...] = jnp.concatenate([smsg] + vm, axis=1)


def _edge_mlp(d2, ev, ssrc, sdst, vps, cen, wid, wr0, br0, wr1, br1,
              wr2, br2, ws0a, ws0b, bs0, ws1, bs1, be):
    ep, h = ssrc.shape
    r = cen.shape[1]
    full = lambda s: pl.BlockSpec(s, lambda i: tuple(0 for _ in s))
    return pl.pallas_call(
        _edge_body,
        grid=(ep // be,),
        in_specs=[
            pl.BlockSpec((be, 1), lambda i: (i, 0)),
            pl.BlockSpec((be, 3), lambda i: (i, 0)),
            pl.BlockSpec((be, h), lambda i: (i, 0)),
            pl.BlockSpec((be, h), lambda i: (i, 0)),
            pl.BlockSpec((be, 3 * h), lambda i: (i, 0)),
            full((1, r)), full((1, r)),
            full((r, h)), full((1, h)),
            full((h, h)), full((1, h)),
            full((h, h)), full((1, h)),
            full((h, h)), full((h, h)), full((1, h)),
            full((h, h)), full((1, h)),
        ],
        out_specs=pl.BlockSpec((be, 4 * h), lambda i: (i, 0)),
        out_shape=jax.ShapeDtypeStruct((ep, 4 * h), jnp.float32),
    )(d2, ev, ssrc, sdst, vps, cen, wid, wr0, br0, wr1, br1, wr2, br2,
      ws0a, ws0b, bs0, ws1, bs1)


def _scatter_body(msgs, dst2, out, idxb, mbuf, acc):
    n = out.shape[0]
    ep = msgs.shape[0]
    rows_t = ep // _CH // 16
    nt = 8 * (n // (16 * 8))
    nrem = n - 16 * nt
    nfull = nt // _CH
    ntail = nt - nfull * _CH
    cid = lax.axis_index("c")
    sid = lax.axis_index("s")
    last = sid == 15
    pltpu.sync_copy(dst2.at[pl.ds(sid * rows_t, rows_t)], idxb)

    for p in range(2):
        col = (cid * 2 + p) * _CH

        def zrow(rr, carry):
            for t in range(8):
                mbuf[rr, pl.ds(t * 16, 16)] = jnp.zeros((16,), jnp.float32)
            return carry

        lax.fori_loop(0, _CH, zrow, 0)
        for j in range(nfull):
            pltpu.sync_copy(mbuf, acc.at[pl.ds(sid * nt + j * _CH, _CH)])
        if ntail:
            pltpu.sync_copy(mbuf.at[pl.ds(0, ntail)],
                            acc.at[pl.ds(sid * nt + nfull * _CH, ntail)])
        if nrem:
            @pl.when(last)
            def _():
                pltpu.sync_copy(mbuf.at[pl.ds(0, nrem)],
                                acc.at[pl.ds(16 * nt, nrem)])
        plsc.subcore_barrier()

        def chunk(i, carry):
            e0 = (sid * rows_t + i) * _CH
            pltpu.sync_copy(msgs.at[pl.ds(e0, _CH), pl.ds(col, _CH)], mbuf)
            pltpu.sync_copy(mbuf, acc.at[idxb.at[i]], add=True)
            return carry

        lax.fori_loop(0, rows_t, chunk, 0)
        plsc.subcore_barrier()
        pltpu.sync_copy(acc.at[pl.ds(sid * nt, nt)],
                        out.at[pl.ds(sid * nt, nt), pl.ds(col, _CH)])
        if nrem:
            @pl.when(last)
            def _():
                pltpu.sync_copy(acc.at[pl.ds(16 * nt, nrem)],
                                out.at[pl.ds(16 * nt, nrem), pl.ds(col, _CH)])
        plsc.subcore_barrier()


def _sc_scatter(msgs, dst2, n):
    ep = msgs.shape[0]
    rows_t = ep // _CH // 16
    mesh = plsc.VectorSubcoreMesh(core_axis_name="c", subcore_axis_name="s", num_cores=2, num_subcores=16)
    f = functools.partial(
        pl.kernel,
        out_type=jax.ShapeDtypeStruct((n, 512), jnp.float32),
        mesh=mesh,
        scratch_types=[
            pltpu.VMEM((rows_t, _CH), jnp.int32),
            pltpu.VMEM((_CH, _CH), jnp.float32),
            pltpu.VMEM_SHARED((n, _CH), jnp.float32),
        ],
    )(_scatter_body)
    return f(msgs, dst2)


def _node_body(s_ref, agg_ref, v3_ref, wu0a_ref, wu0b_ref, bu0_ref,
               wu1_ref, bu1_ref, wvu_ref, gsn_ref, bsn_ref, gvn_ref,
               bvn_ref, so_ref, vo_ref):
    s = s_ref[...]
    ag = agg_ref[...]
    h = s.shape[1]
    u = _silu(jnp.dot(s, wu0a_ref[...], preferred_element_type=jnp.float32)
              + jnp.dot(ag[:, :h], wu0b_ref[...],
                        preferred_element_type=jnp.float32)
              + bu0_ref[...])
    s2 = s + jnp.dot(u, wu1_ref[...], preferred_element_type=jnp.float32) \
        + bu1_ref[...]
    so_ref[...] = _ln(s2, gsn_ref[...], bsn_ref[...])
    wvu = wvu_ref[...]
    outs = []
    for k in range(3):
        t = v3_ref[k] + jnp.dot(ag[:, (k + 1) * h:(k + 2) * h], wvu,
                                preferred_element_type=jnp.float32)
        outs.append(_ln(t, gvn_ref[...], bvn_ref[...]))
    vo_ref[...] = jnp.stack(outs, axis=0)


def _node_update(s, agg, v3, wu0a, wu0b, bu0, wu1, bu1, wvu,
                 gsn, bsn, gvn, bvn, bn):
    n, h = s.shape
    full = lambda sh: pl.BlockSpec(sh, lambda i: tuple(0 for _ in sh))
    return pl.pallas_call(
        _node_body,
        grid=(n // bn,),
        in_specs=[
            pl.BlockSpec((bn, h), lambda i: (i, 0)),
            pl.BlockSpec((bn, 4 * h), lambda i: (i, 0)),
            pl.BlockSpec((3, bn, h), lambda i: (0, i, 0)),
            full((h, h)), full((h, h)), full((1, h)),
            full((h, h)), full((1, h)),
            full((h, h)),
            full((1, h)), full((1, h)), full((1, h)), full((1, h)),
        ],
        out_specs=[
            pl.BlockSpec((bn, h), lambda i: (i, 0)),
            pl.BlockSpec((3, bn, h), lambda i: (0, i, 0)),
        ],
        out_shape=[
            jax.ShapeDtypeStruct((n, h), jnp.float32),
            jax.ShapeDtypeStruct((3, n, h), jnp.float32),
        ],
    )(s, agg, v3, wu0a, wu0b, bu0, wu1, bu1, wvu, gsn, bsn, gvn, bvn)


def kernel(scalar_features, vector_features, edge_index, edge_vec, edge_dist,
           centers, widths, W_r0, b_r0, W_r1, b_r1, W_r2, b_r2, W_s0, b_s0,
           W_s1, b_s1, W_vl, W_u0, b_u0, W_u1, b_u1, W_vu, g_sn, b_sn,
           g_vn, b_vn):
    n, h = scalar_features.shape
    e = edge_index.shape[1]
    nrbf = centers.shape[0]

    ep = ((e + 4095) // 4096) * 4096
    pad = ep - e
    src = jnp.concatenate([edge_index[0], jnp.zeros((pad,), jnp.int32)])
    dst = jnp.concatenate([edge_index[1], jnp.zeros((pad,), jnp.int32)])
    src2 = src.reshape(ep // _CH, _CH)
    dst2 = dst.reshape(ep // _CH, _CH)
    d2 = jnp.concatenate(
        [edge_dist, jnp.full((pad,), 10.0 * _CUTOFF, jnp.float32)]
    ).reshape(ep, 1)
    ev = jnp.concatenate([edge_vec, jnp.zeros((pad, 3), jnp.float32)])

    v3 = vector_features.transpose(2, 0, 1)

    row = lambda x: x.reshape(1, -1)
    vpf = _vl_transform(v3, W_vl, 1000)
    ssrc, sdst, vps = _sc_gather(scalar_features, vpf, src2, dst2)
    msgs = _edge_mlp(d2, ev, ssrc, sdst, vps, row(centers), row(widths),
                     W_r0, row(b_r0), W_r1, row(b_r1), W_r2, row(b_r2),
                     W_s0[:h], W_s0[h:], row(b_s0), W_s1, row(b_s1), 1024)
    agg = _sc_scatter(msgs, dst2, n)
    s_out, v3_out = _node_update(
        scalar_features, agg, v3, W_u0[:h], W_u0[h:], row(b_u0),
        W_u1, row(b_u1), W_vu, row(g_sn), row(b_sn), row(g_vn), row(b_vn),
        1000)
    return (s_out, v3_out.transpose(1, 2, 0))
